# baseline (speedup 1.0000x reference)
import functools

import jax
import jax.numpy as jnp
import numpy as np
from jax import lax
from jax.experimental import pallas as pl
from jax.experimental.pallas import tpu as pltpu
from jax.experimental.pallas import tpu_sc as plsc

_E = 320000
_EPAD = 327680
_ER = 2560
_TPT = 80
_B = 64
_D = 128

_ACC_HALF = 5120
_ACC_R = 5120

_SC_PARAMS = pltpu.CompilerParams(needs_layout_passes=False)
_MESH = plsc.VectorSubcoreMesh(core_axis_name="c", subcore_axis_name="s")


def _i16(v):
    return jnp.full((16,), v, jnp.int32)


_IOTA16 = lambda: lax.broadcasted_iota(jnp.int32, (16,), 0)


def _sc_deg_body(n_pad, src_hbm, dst_hbm,
                 degp_hbm, srcA_hbm, dstA_hbm, cntA_hbm,
                 srcB_hbm, dstB_hbm, cntB_hbm,
                 src_v, dst_v, hist_v, sA, dA, sB, dB, cbuf, sem):
    cid = lax.axis_index("c")
    sid = lax.axis_index("s")
    wid = sid * 2 + cid
    pltpu.sync_copy(src_hbm.at[pl.ds(wid * _TPT, _TPT)], src_v)
    pltpu.sync_copy(dst_hbm.at[pl.ds(wid * _TPT, _TPT)], dst_v)

    def zero(i, _):
        hist_v[pl.ds(i * 16, 16)] = jnp.zeros((16,), jnp.float32)
        return 0
    lax.fori_loop(0, n_pad // 16, zero, 0)

    z16 = jnp.zeros((16,), jnp.int32)
    p16 = _i16(2 * _ACC_HALF - 1)

    def fill(r, _):
        for cc in range(8):
            sA[r, pl.ds(cc * 16, 16)] = p16
            dA[r, pl.ds(cc * 16, 16)] = z16
            sB[r, pl.ds(cc * 16, 16)] = p16
            dB[r, pl.ds(cc * 16, 16)] = z16
        return 0
    lax.fori_loop(0, _TPT + 2, fill, 0)

    ones16 = jnp.ones((16,), jnp.float32)

    def body(r, offs):
        offa, offb = offs
        for cc in range(8):
            s16 = src_v[r, pl.ds(cc * 16, 16)]
            d16 = dst_v[r, pl.ds(cc * 16, 16)]
            plsc.addupdate_scatter(hist_v, [d16], ones16)
            mA = d16 < _ACC_HALF
            mB = jnp.logical_and(d16 >= _ACC_HALF, d16 < 2 * _ACC_HALF)
            pA = offa + plsc.cumsum(mA.astype(jnp.int32)) - 1
            plsc.store_scatter(sA, [pA >> 7, pA & 127], s16, mask=mA)
            plsc.store_scatter(dA, [pA >> 7, pA & 127], d16, mask=mA)
            offa = offa + jnp.sum(mA.astype(jnp.int32))
            pB = offb + plsc.cumsum(mB.astype(jnp.int32)) - 1
            plsc.store_scatter(sB, [pB >> 7, pB & 127], s16, mask=mB)
            plsc.store_scatter(dB, [pB >> 7, pB & 127], d16 - _ACC_HALF,
                               mask=mB)
            offb = offb + jnp.sum(mB.astype(jnp.int32))
        return offa, offb
    offa, offb = lax.fori_loop(0, _TPT, body, (jnp.int32(0), jnp.int32(0)))

    pltpu.sync_copy(hist_v, degp_hbm.at[wid])
    pltpu.sync_copy(sA.at[pl.ds(0, _TPT)], srcA_hbm.at[pl.ds(wid * _TPT, _TPT)])
    pltpu.sync_copy(dA.at[pl.ds(0, _TPT)], dstA_hbm.at[pl.ds(wid * _TPT, _TPT)])
    pltpu.sync_copy(sB.at[pl.ds(0, _TPT)], srcB_hbm.at[pl.ds(wid * _TPT, _TPT)])
    pltpu.sync_copy(dB.at[pl.ds(0, _TPT)], dstB_hbm.at[pl.ds(wid * _TPT, _TPT)])
    cbuf[...] = _i16(0) + ((offa + 511) & -512)
    pltpu.sync_copy(cbuf, cntA_hbm.at[wid])
    cbuf[...] = _i16(0) + ((offb + 511) & -512)
    pltpu.sync_copy(cbuf, cntB_hbm.at[wid])


def _make_sc_deg(n_pad):
    return pl.kernel(
        functools.partial(_sc_deg_body, n_pad),
        out_type=(
            jax.ShapeDtypeStruct((32, n_pad), jnp.float32),
            jax.ShapeDtypeStruct((_ER, 128), jnp.int32),
            jax.ShapeDtypeStruct((_ER, 128), jnp.int32),
            jax.ShapeDtypeStruct((32, 16), jnp.int32),
            jax.ShapeDtypeStruct((_ER, 128), jnp.int32),
            jax.ShapeDtypeStruct((_ER, 128), jnp.int32),
            jax.ShapeDtypeStruct((32, 16), jnp.int32),
        ),
        mesh=_MESH,
        compiler_params=_SC_PARAMS,
        scratch_types=[
            pltpu.VMEM((_TPT, 128), jnp.int32),
            pltpu.VMEM((_TPT, 128), jnp.int32),
            pltpu.VMEM((n_pad,), jnp.float32),
            pltpu.VMEM((_TPT + 2, 128), jnp.int32),
            pltpu.VMEM((_TPT + 2, 128), jnp.int32),
            pltpu.VMEM((_TPT + 2, 128), jnp.int32),
            pltpu.VMEM((_TPT + 2, 128), jnp.int32),
            pltpu.VMEM((16,), jnp.int32),
            pltpu.SemaphoreType.DMA,
        ],
    )


def _sc_agg_body(g_hbm, src_hbm, dst_hbm, cnt_hbm, out_hbm,
                 src_v, dst_v, r0, r1, r2, r3, cntv,
                 g0, g1, g2, g3, acc_sh):
    cid = lax.axis_index("c")
    sid = lax.axis_index("s")
    wid = sid * 2 + cid
    rpt = _ACC_R // 16

    for r in range(128):
        for cc in range(8):
            r0[r, pl.ds(cc * 16, 16)] = jnp.zeros((16,), jnp.float32)
    for b in range(rpt // 32):
        pltpu.sync_copy(r0.at[pl.ds(0, 32)],
                        acc_sh.at[pl.ds(sid * rpt + b * 32, 32)])

    pltpu.sync_copy(src_hbm.at[pl.ds(wid * _TPT, _TPT)], src_v)
    pltpu.sync_copy(dst_hbm.at[pl.ds(wid * _TPT, _TPT)], dst_v)
    pltpu.sync_copy(cnt_hbm.at[wid], cntv)
    nblk = jnp.max(cntv[...]) >> 9
    plsc.subcore_barrier()

    rows = (r0, r1, r2, r3)
    gsem = (g0, g1, g2, g3)

    def blk(i, _):
        j0 = i * 4
        cg = [pltpu.async_copy(g_hbm.at[src_v.at[j0 + q]], rows[q], gsem[q])
              for q in range(4)]
        for q in range(4):
            cg[q].wait()
            pltpu.sync_copy(rows[q], acc_sh.at[dst_v.at[j0 + q]], add=True)
        return 0
    lax.fori_loop(0, nblk, blk, 0)
    plsc.subcore_barrier()

    for off, sz in ((0, 128), (128, 128), (256, 64)):
        sl = pl.ds(sid * rpt + off, sz)
        pltpu.sync_copy(acc_sh.at[sl], r0.at[pl.ds(0, sz)])
        pltpu.sync_copy(r0.at[pl.ds(0, sz)], out_hbm.at[cid, sl])


def _make_sc_agg():
    return pl.kernel(
        _sc_agg_body,
        out_type=jax.ShapeDtypeStruct((2, _ACC_R, 128), jnp.float32),
        mesh=_MESH,
        compiler_params=_SC_PARAMS,
        scratch_types=[
            pltpu.VMEM((_TPT, 128), jnp.int32),
            pltpu.VMEM((_TPT, 128), jnp.int32),
            pltpu.VMEM((128, 128), jnp.float32),
            pltpu.VMEM((128, 128), jnp.float32),
            pltpu.VMEM((128, 128), jnp.float32),
            pltpu.VMEM((128, 128), jnp.float32),
            pltpu.VMEM((16,), jnp.int32),
            pltpu.SemaphoreType.DMA,
            pltpu.SemaphoreType.DMA,
            pltpu.SemaphoreType.DMA,
            pltpu.SemaphoreType.DMA,
            pltpu.VMEM_SHARED((_ACC_R, 128), jnp.float32),
        ],
    )


def _sc_pool_body(n, n_pad, k, k_pad, has_next, *refs):
    if has_next:
        (hp_hbm, keep_hbm, nid_hbm, gate_hbm, batch_hbm, src_hbm, dst_hbm,
         xnew_hbm, bnew_hbm, srcout_hbm, dstout_hbm, cntout_hbm, degp_hbm,
         keep_v, nid_v, gate_v, batch_v, src_v, dst_v, hist_v, sbufc, dbufc,
         ccnt, perm_v, cbuf, gbuf, bbuf, rows16, sem, perm_sh) = refs
    else:
        (hp_hbm, keep_hbm, nid_hbm, gate_hbm, batch_hbm,
         xnew_hbm, bnew_hbm,
         keep_v, nid_v, gate_v, batch_v, perm_v,
         cbuf, gbuf, bbuf, rows16, sem, perm_sh) = refs
    cid = lax.axis_index("c")
    sid = lax.axis_index("s")
    wid = sid * 2 + cid
    rrows = n_pad // 128

    pltpu.sync_copy(keep_hbm, keep_v)
    pltpu.sync_copy(nid_hbm, nid_v)
    pltpu.sync_copy(gate_hbm, gate_v)
    pltpu.sync_copy(batch_hbm, batch_v)

    @pl.when(sid == 0)
    def _phase_a():
        pad16 = _i16(n_pad - 1)

        def zero(i, _):
            perm_v[pl.ds(i * 16, 16)] = pad16
            return 0
        lax.fori_loop(0, k_pad // 16, zero, 0)
        iota = _IOTA16()

        def body(r, _):
            for cc in range(8):
                k16 = keep_v[r, pl.ds(cc * 16, 16)]
                n16 = nid_v[r, pl.ds(cc * 16, 16)]
                iv = r * 128 + cc * 16 + iota
                plsc.store_scatter(perm_v, [n16], iv, mask=k16 == 1)
            return 0
        lax.fori_loop(0, rrows, body, 0)
        pltpu.sync_copy(perm_v, perm_sh)

    if has_next:
        pltpu.sync_copy(src_hbm.at[pl.ds(wid * _TPT, _TPT)], src_v)
        pltpu.sync_copy(dst_hbm.at[pl.ds(wid * _TPT, _TPT)], dst_v)

        def zeroh(i, _):
            hist_v[pl.ds(i * 16, 16)] = jnp.zeros((16,), jnp.float32)
            return 0
        lax.fori_loop(0, k_pad // 16, zeroh, 0)
        z16 = jnp.zeros((16,), jnp.int32)
        p16 = _i16(k_pad - 1)
        k16c = _i16(k)

        def fill(r, _):
            for cc in range(8):
                sbufc[r, pl.ds(cc * 16, 16)] = p16
                dbufc[r, pl.ds(cc * 16, 16)] = z16
            return 0
        lax.fori_loop(0, _TPT + 2, fill, 0)
        ones16 = jnp.ones((16,), jnp.float32)

        def ebody(r, off):
            for cc in range(8):
                s16 = src_v[r, pl.ds(cc * 16, 16)]
                d16 = dst_v[r, pl.ds(cc * 16, 16)]
                ks = plsc.load_gather(keep_v, [s16 >> 7, s16 & 127])
                kd = plsc.load_gather(keep_v, [d16 >> 7, d16 & 127])
                sn = plsc.load_gather(nid_v, [s16 >> 7, s16 & 127])
                dn = plsc.load_gather(nid_v, [d16 >> 7, d16 & 127])
                act = ks & kd
                m = act == 1
                dstf = jnp.where(m, dn, k16c)
                plsc.addupdate_scatter(hist_v, [dstf], ones16)
                pos = off + plsc.cumsum(act) - 1
                plsc.store_scatter(sbufc, [pos >> 7, pos & 127], sn, mask=m)
                plsc.store_scatter(dbufc, [pos >> 7, pos & 127], dn, mask=m)
                off = off + jnp.sum(act)
            return off
        off = lax.fori_loop(0, _TPT, ebody, jnp.int32(0))
        pltpu.sync_copy(sbufc.at[pl.ds(0, _TPT)],
                        srcout_hbm.at[pl.ds(wid * _TPT, _TPT)])
        pltpu.sync_copy(dbufc.at[pl.ds(0, _TPT)],
                        dstout_hbm.at[pl.ds(wid * _TPT, _TPT)])
        ccnt[...] = _i16(0) + ((off + 511) & -512)
        pltpu.sync_copy(ccnt, cntout_hbm.at[wid])
        pltpu.sync_copy(hist_v, degp_hbm.at[wid])

    plsc.subcore_barrier()

    units = k_pad // 16
    nunits_w = (units - wid + 31) // 32

    def ubody(i, _):
        u = wid + i * 32
        pltpu.sync_copy(perm_sh.at[pl.ds(u * 16, 16)], cbuf)
        idxv = cbuf[...]
        pltpu.async_copy(hp_hbm.at[idxv], rows16, sem).wait()
        gbuf[...] = plsc.load_gather(gate_v, [idxv >> 7, idxv & 127])
        bbuf[...] = plsc.load_gather(batch_v, [idxv >> 7, idxv & 127])
        for e in range(16):
            bc = plsc.load_gather(gbuf, [_i16(e)])
            for cc in range(8):
                rows16[e, pl.ds(cc * 16, 16)] = (
                    rows16[e, pl.ds(cc * 16, 16)] * bc)
        pltpu.sync_copy(rows16, xnew_hbm.at[pl.ds(u * 16, 16)])
        pltpu.sync_copy(bbuf, bnew_hbm.at[u >> 3, pl.ds((u & 7) * 16, 16)])
        return 0
    lax.fori_loop(0, nunits_w, ubody, 0)


def _make_sc_pool(n, n_pad, k, k_pad, has_next):
    rrows = n_pad // 128
    outs = [
        jax.ShapeDtypeStruct((k_pad, 128), jnp.float32),
        jax.ShapeDtypeStruct((k_pad // 128, 128), jnp.int32),
    ]
    scratch = [
        pltpu.VMEM((rrows, 128), jnp.int32),
        pltpu.VMEM((rrows, 128), jnp.int32),
        pltpu.VMEM((rrows, 128), jnp.float32),
        pltpu.VMEM((rrows, 128), jnp.int32),
    ]
    if has_next:
        outs += [
            jax.ShapeDtypeStruct((_ER, 128), jnp.int32),
            jax.ShapeDtypeStruct((_ER, 128), jnp.int32),
            jax.ShapeDtypeStruct((32, 16), jnp.int32),
            jax.ShapeDtypeStruct((32, k_pad), jnp.float32),
        ]
        scratch += [
            pltpu.VMEM((_TPT, 128), jnp.int32),
            pltpu.VMEM((_TPT, 128), jnp.int32),
            pltpu.VMEM((k_pad,), jnp.float32),
            pltpu.VMEM((_TPT + 2, 128), jnp.int32),
            pltpu.VMEM((_TPT + 2, 128), jnp.int32),
            pltpu.VMEM((16,), jnp.int32),
        ]
    scratch += [
        pltpu.VMEM((k_pad,), jnp.int32),
        pltpu.VMEM((16,), jnp.int32),
        pltpu.VMEM((16,), jnp.float32),
        pltpu.VMEM((16,), jnp.int32),
        pltpu.VMEM((16, 128), jnp.float32),
        pltpu.SemaphoreType.DMA,
        pltpu.VMEM_SHARED((k_pad,), jnp.int32),
    ]
    return pl.kernel(
        functools.partial(_sc_pool_body, n, n_pad, k, k_pad, has_next),
        out_type=tuple(outs),
        mesh=_MESH,
        compiler_params=_SC_PARAMS,
        scratch_types=scratch,
    )


def _tc_prep_body(h_ref, w_ref, degp_ref, g_ref, dinv_ref):
    ones = jnp.ones((32, 1), jnp.float32)
    deg = lax.dot_general(degp_ref[...], ones, (((0,), (0,)), ((), ())),
                          preferred_element_type=jnp.float32) + 1.0
    dinv = 1.0 / jnp.sqrt(deg)
    hw = jnp.dot(h_ref[...], w_ref[...], preferred_element_type=jnp.float32)
    g_ref[...] = hw * dinv
    dinv_ref[...] = dinv


def _tc_prep(h, w, degp):
    n_pad = h.shape[0]
    return pl.pallas_call(
        _tc_prep_body,
        out_shape=(jax.ShapeDtypeStruct((n_pad, 128), jnp.float32),
                   jax.ShapeDtypeStruct((n_pad, 1), jnp.float32)),
    )(h, w, degp)


def _tc_post_a_body(two_pass, n_pad, *refs):
    if two_pass:
        slo_ref, shi_ref, g_ref, dinv_ref, b_ref, ws_ref, hp_ref, sc_ref = refs
        lo = (slo_ref[0] + slo_ref[1])[:_ACC_HALF]
        hi = (shi_ref[0] + shi_ref[1])[:_ACC_HALF]
        s = jnp.concatenate([lo, hi], axis=0)
    else:
        slo_ref, g_ref, dinv_ref, b_ref, ws_ref, hp_ref, sc_ref = refs
        s = (slo_ref[0] + slo_ref[1])[:n_pad]
    s = s + g_ref[...]
    hp = jnp.maximum(s * dinv_ref[...] + b_ref[...], 0.0)
    hp_ref[...] = hp
    sc_ref[...] = jnp.dot(hp, ws_ref[...], preferred_element_type=jnp.float32)


def _tc_post_a(s_parts, g, dinv, b, ws):
    n_pad = g.shape[0]
    two_pass = len(s_parts) == 2
    return pl.pallas_call(
        functools.partial(_tc_post_a_body, two_pass, n_pad),
        out_shape=(jax.ShapeDtypeStruct((n_pad, 128), jnp.float32),
                   jax.ShapeDtypeStruct((n_pad, 1), jnp.float32)),
    )(*s_parts, g, dinv, b, ws)


def _tc_post_b_body(n, k, score_ref, keep_ref, nid_ref, gate_ref):
    rr = score_ref.shape[0]
    score = score_ref[...]
    flat = (lax.broadcasted_iota(jnp.int32, (rr, 128), 0) * 128
            + lax.broadcasted_iota(jnp.int32, (rr, 128), 1))
    valid = flat < n
    ikey = lax.bitcast_convert_type(score, jnp.int32)
    key = ikey ^ ((ikey >> 31) & jnp.int32(0x7FFFFFFF))
    uk = lax.bitcast_convert_type(key ^ jnp.int32(-2147483648), jnp.uint32)
    uk = jnp.where(valid, uk, jnp.uint32(0))

    def sbody(i, t):
        cand = t | (jnp.uint32(1) << (jnp.uint32(31) - i.astype(jnp.uint32)))
        cnt = jnp.sum((uk >= cand).astype(jnp.int32))
        return jnp.where(cnt >= k, cand, t)
    tthr = lax.fori_loop(0, 32, sbody, jnp.uint32(0))

    gt = uk > tthr
    eq = jnp.logical_and(uk == tthr, valid)
    needed = (k - jnp.sum(gt.astype(jnp.int32))).astype(jnp.float32)

    iu = lax.broadcasted_iota(jnp.int32, (128, 128), 0)
    ju = lax.broadcasted_iota(jnp.int32, (128, 128), 1)
    tri = (iu <= ju).astype(jnp.float32)
    ir = lax.broadcasted_iota(jnp.int32, (rr, rr), 0)
    jr = lax.broadcasted_iota(jnp.int32, (rr, rr), 1)
    ltri = (jr < ir).astype(jnp.float32)

    def cumsum2d(x):
        p = jnp.dot(x, tri, preferred_element_type=jnp.float32)
        tot = p[:, 127:128]
        off = jnp.dot(ltri, tot, preferred_element_type=jnp.float32)
        return p + off

    eqf = eq.astype(jnp.float32)
    rank = cumsum2d(eqf) - eqf
    keep = jnp.logical_or(gt, jnp.logical_and(eq, rank < needed))
    keep = jnp.logical_and(keep, valid)
    keepf = keep.astype(jnp.float32)
    incl = cumsum2d(keepf)
    nid = jnp.clip(incl - 1.0, 0.0, float(k - 1)).astype(jnp.int32)
    keep_ref[...] = keep.astype(jnp.int32)
    nid_ref[...] = nid
    gate_ref[...] = jnp.where(valid, jax.nn.sigmoid(score), 0.0)


def _tc_post_b(score2d, n, k):
    rr = score2d.shape[0]
    return pl.pallas_call(
        functools.partial(_tc_post_b_body, n, k),
        out_shape=(jax.ShapeDtypeStruct((rr, 128), jnp.int32),
                   jax.ShapeDtypeStruct((rr, 128), jnp.int32),
                   jax.ShapeDtypeStruct((rr, 128), jnp.float32)),
    )(score2d)


def _tc_readout_body(k, final, *refs):
    if final:
        (x_ref, bcol_ref, prev_ref, l1w_ref, l1b_ref, l2w_ref, l2b_ref,
         out_ref) = refs
    else:
        x_ref, bcol_ref, prev_ref, out_ref = refs
    kp = x_ref.shape[0]
    x = x_ref[...]
    bcol = bcol_ref[...]
    validc = lax.broadcasted_iota(jnp.int32, (kp, 1), 0) < k
    gids = lax.broadcasted_iota(jnp.int32, (1, _B), 1)
    onehot = jnp.logical_and(bcol == gids, validc).astype(jnp.float32)
    cnt = jnp.sum(onehot, axis=0, keepdims=True)
    sums = lax.dot_general(onehot, x, (((0,), (0,)), ((), ())),
                           preferred_element_type=jnp.float32)
    cntc = cnt.reshape(_B, 1)
    mean = sums / jnp.maximum(cntc, 1.0)
    neg = jnp.float32(-3.0e38)
    mxs = []
    for g in range(_B):
        m = onehot[:, g:g + 1] > 0.0
        mxs.append(jnp.max(jnp.where(m, x, neg), axis=0, keepdims=True))
    mx = jnp.concatenate(mxs, axis=0)
    mx = jnp.where(cntc > 0.0, mx, 0.0)
    z = prev_ref[...] + jnp.concatenate([mx, mean], axis=1)
    if final:
        zz = jnp.maximum(
            jnp.dot(z, l1w_ref[...], preferred_element_type=jnp.float32)
            + l1b_ref[...], 0.0)
        out_ref[...] = jax.nn.sigmoid(
            jnp.dot(zz, l2w_ref[...], preferred_element_type=jnp.float32)
            + l2b_ref[...])
    else:
        out_ref[...] = z


def _tc_readout(x, bcol, prev, k):
    return pl.pallas_call(
        functools.partial(_tc_readout_body, k, False),
        out_shape=jax.ShapeDtypeStruct((_B, 2 * _D), jnp.float32),
    )(x, bcol, prev)


def _tc_readout_final(x, bcol, prev, k, l1w, l1b, l2w, l2b):
    return pl.pallas_call(
        functools.partial(_tc_readout_body, k, True),
        out_shape=jax.ShapeDtypeStruct((_B, 1), jnp.float32),
    )(x, bcol, prev, l1w, l1b, l2w, l2b)


_L = [
    dict(n=10000, n_pad=10240, k=5000, k_pad=5120),
    dict(n=5000, n_pad=5120, k=2500, k_pad=2560),
    dict(n=2500, n_pad=2560, k=1250, k_pad=1280),
]

_sc_deg1 = _make_sc_deg(_L[0]["n_pad"])
_AGG_PAD = _L[0]["n_pad"]
_sc_agg1 = _make_sc_agg()
_sc_pools = [_make_sc_pool(p["n"], p["n_pad"], p["k"], p["k_pad"], i < 2)
             for i, p in enumerate(_L)]


def kernel(x, pos, edge_index, edge_attr, strata_data, batch, W1, b1, W2, b2,
           W3, b3, ws1, ws2, ws3, lin1_w, lin1_b, lin2_w, lin2_b):
    p1, p2, p3 = _L
    src = jnp.concatenate(
        [edge_index[0], jnp.zeros((_EPAD - _E,), jnp.int32)]).reshape(_ER, 128)
    dst = jnp.concatenate(
        [edge_index[1], jnp.full((_EPAD - _E,), p1["n"], jnp.int32)]
    ).reshape(_ER, 128)
    h0 = jnp.pad(jnp.concatenate([x, pos], axis=1),
                 ((0, p1["n_pad"] - p1["n"]), (0, 0)))
    batch2d = jnp.pad(batch, (0, p1["n_pad"] - p1["n"])).reshape(-1, 128)

    ws = [ws1, ws2, ws3]
    Ws = [W1, W2, W3]
    bs = [b1.reshape(1, -1), b2.reshape(1, -1), b3.reshape(1, -1)]

    z = jnp.zeros((_B, 2 * _D), jnp.float32)
    h = h0
    degp, srcA, dstA, cntA, srcB, dstB, cntB = _sc_deg1(src, dst)
    elists = [(srcA, dstA, cntA), (srcB, dstB, cntB)]
    out = None
    for t, pt in enumerate(_L):
        g, dinv = _tc_prep(h, Ws[t], degp)
        gp = g if pt["n_pad"] == _AGG_PAD else jnp.pad(
            g, ((0, _AGG_PAD - pt["n_pad"]), (0, 0)))
        s_parts = [_sc_agg1(gp, es, ed, ec) for es, ed, ec in elists]
        hp, score = _tc_post_a(s_parts, g, dinv, bs[t], ws[t])
        keep2d, nid2d, gate2d = _tc_post_b(
            score.reshape(-1, 128), pt["n"], pt["k"])
        if t < 2:
            xnew, bnew, src, dst, cnt, degp = _sc_pools[t](
                hp, keep2d, nid2d, gate2d, batch2d, src, dst)
            elists = [(src, dst, cnt)]
        else:
            xnew, bnew = _sc_pools[t](hp, keep2d, nid2d, gate2d, batch2d)
        bcol = bnew.reshape(-1, 1)
        if t < 2:
            z = _tc_readout(xnew, bcol, z, pt["k"])
        else:
            out = _tc_readout_final(
                xnew, bcol, z, pt["k"], lin1_w, lin1_b.reshape(1, -1),
                lin2_w, lin2_b.reshape(1, -1))
        h = xnew
        batch2d = bnew
    return out


# speedup vs baseline: 1.4704x; 1.4704x over previous
import functools

import jax
import jax.numpy as jnp
import numpy as np
from jax import lax
from jax.experimental import pallas as pl
from jax.experimental.pallas import tpu as pltpu
from jax.experimental.pallas import tpu_sc as plsc

_E = 320000
_EPAD = 327680
_ER = 2560
_TPT = 80
_B = 64
_D = 128

_ACC_HALF = 5120
_ACC_R = 5632

_SC_PARAMS = pltpu.CompilerParams(needs_layout_passes=False)
_MESH = plsc.VectorSubcoreMesh(core_axis_name="c", subcore_axis_name="s")


def _i16(v):
    return jnp.full((16,), v, jnp.int32)


_IOTA16 = lambda: lax.broadcasted_iota(jnp.int32, (16,), 0)


def _sc_deg_body(n_pad, src_hbm, dst_hbm,
                 degp_hbm, srcA_hbm, dstA_hbm, cntA_hbm,
                 srcB_hbm, dstB_hbm, cntB_hbm,
                 src_v, dst_v, hist_v, sA, dA, sB, dB, cbuf, sem):
    cid = lax.axis_index("c")
    sid = lax.axis_index("s")
    wid = sid * 2 + cid
    pltpu.sync_copy(src_hbm.at[pl.ds(wid * _TPT, _TPT)], src_v)
    pltpu.sync_copy(dst_hbm.at[pl.ds(wid * _TPT, _TPT)], dst_v)

    def zero(i, _):
        hist_v[pl.ds(i * 16, 16)] = jnp.zeros((16,), jnp.float32)
        return 0
    lax.fori_loop(0, n_pad // 16, zero, 0)

    z16 = jnp.zeros((16,), jnp.int32)
    t16 = _i16(_ACC_HALF)

    def fill(r, _):
        for cc in range(8):
            sA[r, pl.ds(cc * 16, 16)] = z16
            dA[r, pl.ds(cc * 16, 16)] = t16
            sB[r, pl.ds(cc * 16, 16)] = z16
            dB[r, pl.ds(cc * 16, 16)] = t16
        return 0
    lax.fori_loop(0, _TPT + 2, fill, 0)

    ones16 = jnp.ones((16,), jnp.float32)

    def body(r, offs):
        offa, offb = offs
        for cc in range(8):
            s16 = src_v[r, pl.ds(cc * 16, 16)]
            d16 = dst_v[r, pl.ds(cc * 16, 16)]
            plsc.addupdate_scatter(hist_v, [d16], ones16)
            mA = d16 < _ACC_HALF
            mB = jnp.logical_and(d16 >= _ACC_HALF, d16 < 2 * _ACC_HALF)
            pA = offa + plsc.cumsum(mA.astype(jnp.int32)) - 1
            plsc.store_scatter(sA, [pA >> 7, pA & 127], s16, mask=mA)
            plsc.store_scatter(dA, [pA >> 7, pA & 127], d16, mask=mA)
            offa = offa + jnp.sum(mA.astype(jnp.int32))
            pB = offb + plsc.cumsum(mB.astype(jnp.int32)) - 1
            plsc.store_scatter(sB, [pB >> 7, pB & 127], s16, mask=mB)
            plsc.store_scatter(dB, [pB >> 7, pB & 127], d16 - _ACC_HALF,
                               mask=mB)
            offb = offb + jnp.sum(mB.astype(jnp.int32))
        return offa, offb
    offa, offb = lax.fori_loop(0, _TPT, body, (jnp.int32(0), jnp.int32(0)))

    pltpu.sync_copy(hist_v, degp_hbm.at[wid])
    pltpu.sync_copy(sA.at[pl.ds(0, _TPT)], srcA_hbm.at[pl.ds(wid * _TPT, _TPT)])
    pltpu.sync_copy(dA.at[pl.ds(0, _TPT)], dstA_hbm.at[pl.ds(wid * _TPT, _TPT)])
    pltpu.sync_copy(sB.at[pl.ds(0, _TPT)], srcB_hbm.at[pl.ds(wid * _TPT, _TPT)])
    pltpu.sync_copy(dB.at[pl.ds(0, _TPT)], dstB_hbm.at[pl.ds(wid * _TPT, _TPT)])
    cbuf[...] = _i16(0) + ((offa + 255) & -256)
    pltpu.sync_copy(cbuf, cntA_hbm.at[wid])
    cbuf[...] = _i16(0) + ((offb + 255) & -256)
    pltpu.sync_copy(cbuf, cntB_hbm.at[wid])


def _make_sc_deg(n_pad):
    return pl.kernel(
        functools.partial(_sc_deg_body, n_pad),
        out_type=(
            jax.ShapeDtypeStruct((32, n_pad), jnp.float32),
            jax.ShapeDtypeStruct((_ER, 128), jnp.int32),
            jax.ShapeDtypeStruct((_ER, 128), jnp.int32),
            jax.ShapeDtypeStruct((32, 16), jnp.int32),
            jax.ShapeDtypeStruct((_ER, 128), jnp.int32),
            jax.ShapeDtypeStruct((_ER, 128), jnp.int32),
            jax.ShapeDtypeStruct((32, 16), jnp.int32),
        ),
        mesh=_MESH,
        compiler_params=_SC_PARAMS,
        scratch_types=[
            pltpu.VMEM((_TPT, 128), jnp.int32),
            pltpu.VMEM((_TPT, 128), jnp.int32),
            pltpu.VMEM((n_pad,), jnp.float32),
            pltpu.VMEM((_TPT + 2, 128), jnp.int32),
            pltpu.VMEM((_TPT + 2, 128), jnp.int32),
            pltpu.VMEM((_TPT + 2, 128), jnp.int32),
            pltpu.VMEM((_TPT + 2, 128), jnp.int32),
            pltpu.VMEM((16,), jnp.int32),
            pltpu.SemaphoreType.DMA,
        ],
    )


def _sc_agg_body(g_hbm, src_hbm, dst_hbm, cnt_hbm, out_hbm,
                 src_v, dst_v, rows_a, rows_b, zbuf,
                 cntv, sem_a, sem_b, sem_z, acc_sh):
    cid = lax.axis_index("c")
    sid = lax.axis_index("s")
    wid = sid * 2 + cid
    rpt = _ACC_R // 16

    for r in range(32):
        for cc in range(8):
            zbuf[r, pl.ds(cc * 16, 16)] = jnp.zeros((16,), jnp.float32)
    for b in range(rpt // 32):
        pltpu.sync_copy(zbuf, acc_sh.at[pl.ds(sid * rpt + b * 32, 32)])

    pltpu.sync_copy(src_hbm.at[pl.ds(wid * _TPT, _TPT)], src_v)
    pltpu.sync_copy(dst_hbm.at[pl.ds(wid * _TPT, _TPT)], dst_v)
    pltpu.sync_copy(cnt_hbm.at[wid], cntv)
    nsteps = jnp.max(cntv[...]) >> 8
    plsc.subcore_barrier()

    def step(i, _):
        j0 = i * 2
        cp_a = pltpu.async_copy(g_hbm.at[src_v.at[j0]], rows_a, sem_a)
        cp_b = pltpu.async_copy(g_hbm.at[src_v.at[j0 + 1]], rows_b, sem_b)
        cp_a.wait()
        pltpu.sync_copy(rows_a, acc_sh.at[dst_v.at[j0]], add=True)
        cp_b.wait()
        pltpu.sync_copy(rows_b, acc_sh.at[dst_v.at[j0 + 1]], add=True)
        return 0
    lax.fori_loop(0, nsteps, step, 0)
    plsc.subcore_barrier()

    for b in range(rpt // 32):
        sl = pl.ds(sid * rpt + b * 32, 32)
        pltpu.sync_copy(acc_sh.at[sl], zbuf)
        pltpu.sync_copy(zbuf, out_hbm.at[cid, sl])


def _make_sc_agg():
    return pl.kernel(
        _sc_agg_body,
        out_type=jax.ShapeDtypeStruct((2, _ACC_R, 128), jnp.float32),
        mesh=_MESH,
        compiler_params=_SC_PARAMS,
        scratch_types=[
            pltpu.VMEM((_TPT, 128), jnp.int32),
            pltpu.VMEM((_TPT, 128), jnp.int32),
            pltpu.VMEM((128, 128), jnp.float32),
            pltpu.VMEM((128, 128), jnp.float32),
            pltpu.VMEM((32, 128), jnp.float32),
            pltpu.VMEM((16,), jnp.int32),
            pltpu.SemaphoreType.DMA,
            pltpu.SemaphoreType.DMA,
            pltpu.SemaphoreType.DMA,
            pltpu.VMEM_SHARED((_ACC_R, 128), jnp.float32),
        ],
    )


def _sc_pool_body(n, n_pad, k, k_pad, has_next, *refs):
    if has_next:
        (hp_hbm, keep_hbm, nid_hbm, gate_hbm, batch_hbm, src_hbm, dst_hbm,
         xnew_hbm, bnew_hbm, srcout_hbm, dstout_hbm, cntout_hbm, degp_hbm,
         keep_v, nid_v, gate_v, batch_v, src_v, dst_v, hist_v, sbufc, dbufc,
         ccnt, perm_v, cbuf, gbuf, bbuf, rows16, sem, perm_sh) = refs
    else:
        (hp_hbm, keep_hbm, nid_hbm, gate_hbm, batch_hbm,
         xnew_hbm, bnew_hbm,
         keep_v, nid_v, gate_v, batch_v, perm_v,
         cbuf, gbuf, bbuf, rows16, sem, perm_sh) = refs
    cid = lax.axis_index("c")
    sid = lax.axis_index("s")
    wid = sid * 2 + cid
    rrows = n_pad // 128

    pltpu.sync_copy(keep_hbm, keep_v)
    pltpu.sync_copy(nid_hbm, nid_v)
    pltpu.sync_copy(gate_hbm, gate_v)
    pltpu.sync_copy(batch_hbm, batch_v)

    @pl.when(sid == 0)
    def _phase_a():
        def zero(i, _):
            perm_v[pl.ds(i * 16, 16)] = jnp.zeros((16,), jnp.int32)
            return 0
        lax.fori_loop(0, k_pad // 16, zero, 0)
        iota = _IOTA16()

        def body(r, _):
            for cc in range(8):
                k16 = keep_v[r, pl.ds(cc * 16, 16)]
                n16 = nid_v[r, pl.ds(cc * 16, 16)]
                iv = r * 128 + cc * 16 + iota
                plsc.store_scatter(perm_v, [n16], iv, mask=k16 == 1)
            return 0
        lax.fori_loop(0, rrows, body, 0)
        pltpu.sync_copy(perm_v, perm_sh)

    if has_next:
        pltpu.sync_copy(src_hbm.at[pl.ds(wid * _TPT, _TPT)], src_v)
        pltpu.sync_copy(dst_hbm.at[pl.ds(wid * _TPT, _TPT)], dst_v)

        def zeroh(i, _):
            hist_v[pl.ds(i * 16, 16)] = jnp.zeros((16,), jnp.float32)
            return 0
        lax.fori_loop(0, k_pad // 16, zeroh, 0)
        z16 = jnp.zeros((16,), jnp.int32)
        k16c = _i16(k)

        def fill(r, _):
            for cc in range(8):
                sbufc[r, pl.ds(cc * 16, 16)] = z16
                dbufc[r, pl.ds(cc * 16, 16)] = k16c
            return 0
        lax.fori_loop(0, _TPT + 2, fill, 0)
        ones16 = jnp.ones((16,), jnp.float32)

        def ebody(r, off):
            for cc in range(8):
                s16 = src_v[r, pl.ds(cc * 16, 16)]
                d16 = dst_v[r, pl.ds(cc * 16, 16)]
                ks = plsc.load_gather(keep_v, [s16 >> 7, s16 & 127])
                kd = plsc.load_gather(keep_v, [d16 >> 7, d16 & 127])
                sn = plsc.load_gather(nid_v, [s16 >> 7, s16 & 127])
                dn = plsc.load_gather(nid_v, [d16 >> 7, d16 & 127])
                act = ks & kd
                m = act == 1
                dstf = jnp.where(m, dn, k16c)
                plsc.addupdate_scatter(hist_v, [dstf], ones16)
                pos = off + plsc.cumsum(act) - 1
                plsc.store_scatter(sbufc, [pos >> 7, pos & 127], sn, mask=m)
                plsc.store_scatter(dbufc, [pos >> 7, pos & 127], dn, mask=m)
                off = off + jnp.sum(act)
            return off
        off = lax.fori_loop(0, _TPT, ebody, jnp.int32(0))
        pltpu.sync_copy(sbufc.at[pl.ds(0, _TPT)],
                        srcout_hbm.at[pl.ds(wid * _TPT, _TPT)])
        pltpu.sync_copy(dbufc.at[pl.ds(0, _TPT)],
                        dstout_hbm.at[pl.ds(wid * _TPT, _TPT)])
        ccnt[...] = _i16(0) + ((off + 255) & -256)
        pltpu.sync_copy(ccnt, cntout_hbm.at[wid])
        pltpu.sync_copy(hist_v, degp_hbm.at[wid])

    plsc.subcore_barrier()

    units = k_pad // 16
    nunits_w = (units - wid + 31) // 32

    def ubody(i, _):
        u = wid + i * 32
        pltpu.sync_copy(perm_sh.at[pl.ds(u * 16, 16)], cbuf)
        idxv = cbuf[...]
        pltpu.async_copy(hp_hbm.at[idxv], rows16, sem).wait()
        gbuf[...] = plsc.load_gather(gate_v, [idxv >> 7, idxv & 127])
        bbuf[...] = plsc.load_gather(batch_v, [idxv >> 7, idxv & 127])
        for e in range(16):
            bc = plsc.load_gather(gbuf, [_i16(e)])
            for cc in range(8):
                rows16[e, pl.ds(cc * 16, 16)] = (
                    rows16[e, pl.ds(cc * 16, 16)] * bc)
        pltpu.sync_copy(rows16, xnew_hbm.at[pl.ds(u * 16, 16)])
        pltpu.sync_copy(bbuf, bnew_hbm.at[u >> 3, pl.ds((u & 7) * 16, 16)])
        return 0
    lax.fori_loop(0, nunits_w, ubody, 0)


def _make_sc_pool(n, n_pad, k, k_pad, has_next):
    rrows = n_pad // 128
    outs = [
        jax.ShapeDtypeStruct((k_pad, 128), jnp.float32),
        jax.ShapeDtypeStruct((k_pad // 128, 128), jnp.int32),
    ]
    scratch = [
        pltpu.VMEM((rrows, 128), jnp.int32),
        pltpu.VMEM((rrows, 128), jnp.int32),
        pltpu.VMEM((rrows, 128), jnp.float32),
        pltpu.VMEM((rrows, 128), jnp.int32),
    ]
    if has_next:
        outs += [
            jax.ShapeDtypeStruct((_ER, 128), jnp.int32),
            jax.ShapeDtypeStruct((_ER, 128), jnp.int32),
            jax.ShapeDtypeStruct((32, 16), jnp.int32),
            jax.ShapeDtypeStruct((32, k_pad), jnp.float32),
        ]
        scratch += [
            pltpu.VMEM((_TPT, 128), jnp.int32),
            pltpu.VMEM((_TPT, 128), jnp.int32),
            pltpu.VMEM((k_pad,), jnp.float32),
            pltpu.VMEM((_TPT + 2, 128), jnp.int32),
            pltpu.VMEM((_TPT + 2, 128), jnp.int32),
            pltpu.VMEM((16,), jnp.int32),
        ]
    scratch += [
        pltpu.VMEM((k_pad,), jnp.int32),
        pltpu.VMEM((16,), jnp.int32),
        pltpu.VMEM((16,), jnp.float32),
        pltpu.VMEM((16,), jnp.int32),
        pltpu.VMEM((16, 128), jnp.float32),
        pltpu.SemaphoreType.DMA,
        pltpu.VMEM_SHARED((k_pad,), jnp.int32),
    ]
    return pl.kernel(
        functools.partial(_sc_pool_body, n, n_pad, k, k_pad, has_next),
        out_type=tuple(outs),
        mesh=_MESH,
        compiler_params=_SC_PARAMS,
        scratch_types=scratch,
    )


def _tc_prep_body(h_ref, w_ref, degp_ref, g_ref, dinv_ref):
    ones = jnp.ones((32, 1), jnp.float32)
    deg = lax.dot_general(degp_ref[...], ones, (((0,), (0,)), ((), ())),
                          preferred_element_type=jnp.float32) + 1.0
    dinv = 1.0 / jnp.sqrt(deg)
    hw = jnp.dot(h_ref[...], w_ref[...], preferred_element_type=jnp.float32)
    g_ref[...] = hw * dinv
    dinv_ref[...] = dinv


def _tc_prep(h, w, degp):
    n_pad = h.shape[0]
    return pl.pallas_call(
        _tc_prep_body,
        out_shape=(jax.ShapeDtypeStruct((n_pad, 128), jnp.float32),
                   jax.ShapeDtypeStruct((n_pad, 1), jnp.float32)),
    )(h, w, degp)


def _tc_post_a_body(two_pass, n_pad, *refs):
    if two_pass:
        slo_ref, shi_ref, g_ref, dinv_ref, b_ref, ws_ref, hp_ref, sc_ref = refs
        lo = (slo_ref[0] + slo_ref[1])[:_ACC_HALF]
        hi = (shi_ref[0] + shi_ref[1])[:_ACC_HALF]
        s = jnp.concatenate([lo, hi], axis=0)
    else:
        slo_ref, g_ref, dinv_ref, b_ref, ws_ref, hp_ref, sc_ref = refs
        s = (slo_ref[0] + slo_ref[1])[:n_pad]
    s = s + g_ref[...]
    hp = jnp.maximum(s * dinv_ref[...] + b_ref[...], 0.0)
    hp_ref[...] = hp
    sc_ref[...] = jnp.dot(hp, ws_ref[...], preferred_element_type=jnp.float32)


def _tc_post_a(s_parts, g, dinv, b, ws):
    n_pad = g.shape[0]
    two_pass = len(s_parts) == 2
    return pl.pallas_call(
        functools.partial(_tc_post_a_body, two_pass, n_pad),
        out_shape=(jax.ShapeDtypeStruct((n_pad, 128), jnp.float32),
                   jax.ShapeDtypeStruct((n_pad, 1), jnp.float32)),
    )(*s_parts, g, dinv, b, ws)


def _tc_post_b_body(n, k, score_ref, keep_ref, nid_ref, gate_ref):
    rr = score_ref.shape[0]
    score = score_ref[...]
    flat = (lax.broadcasted_iota(jnp.int32, (rr, 128), 0) * 128
            + lax.broadcasted_iota(jnp.int32, (rr, 128), 1))
    valid = flat < n
    ikey = lax.bitcast_convert_type(score, jnp.int32)
    key = ikey ^ ((ikey >> 31) & jnp.int32(0x7FFFFFFF))
    uk = lax.bitcast_convert_type(key ^ jnp.int32(-2147483648), jnp.uint32)
    uk = jnp.where(valid, uk, jnp.uint32(0))

    def sbody(i, t):
        cand = t | (jnp.uint32(1) << (jnp.uint32(31) - i.astype(jnp.uint32)))
        cnt = jnp.sum((uk >= cand).astype(jnp.int32))
        return jnp.where(cnt >= k, cand, t)
    tthr = lax.fori_loop(0, 32, sbody, jnp.uint32(0))

    gt = uk > tthr
    eq = jnp.logical_and(uk == tthr, valid)
    needed = (k - jnp.sum(gt.astype(jnp.int32))).astype(jnp.float32)

    iu = lax.broadcasted_iota(jnp.int32, (128, 128), 0)
    ju = lax.broadcasted_iota(jnp.int32, (128, 128), 1)
    tri = (iu <= ju).astype(jnp.float32)
    ir = lax.broadcasted_iota(jnp.int32, (rr, rr), 0)
    jr = lax.broadcasted_iota(jnp.int32, (rr, rr), 1)
    ltri = (jr < ir).astype(jnp.float32)

    def cumsum2d(x):
        p = jnp.dot(x, tri, preferred_element_type=jnp.float32)
        tot = p[:, 127:128]
        off = jnp.dot(ltri, tot, preferred_element_type=jnp.float32)
        return p + off

    eqf = eq.astype(jnp.float32)
    rank = cumsum2d(eqf) - eqf
    keep = jnp.logical_or(gt, jnp.logical_and(eq, rank < needed))
    keep = jnp.logical_and(keep, valid)
    keepf = keep.astype(jnp.float32)
    incl = cumsum2d(keepf)
    nid = jnp.clip(incl - 1.0, 0.0, float(k - 1)).astype(jnp.int32)
    keep_ref[...] = keep.astype(jnp.int32)
    nid_ref[...] = nid
    gate_ref[...] = jax.nn.sigmoid(score)


def _tc_post_b(score2d, n, k):
    rr = score2d.shape[0]
    return pl.pallas_call(
        functools.partial(_tc_post_b_body, n, k),
        out_shape=(jax.ShapeDtypeStruct((rr, 128), jnp.int32),
                   jax.ShapeDtypeStruct((rr, 128), jnp.int32),
                   jax.ShapeDtypeStruct((rr, 128), jnp.float32)),
    )(score2d)


def _tc_readout_body(k, final, *refs):
    if final:
        (x_ref, bcol_ref, prev_ref, l1w_ref, l1b_ref, l2w_ref, l2b_ref,
         out_ref) = refs
    else:
        x_ref, bcol_ref, prev_ref, out_ref = refs
    kp = x_ref.shape[0]
    x = x_ref[...]
    bcol = bcol_ref[...]
    validc = lax.broadcasted_iota(jnp.int32, (kp, 1), 0) < k
    gids = lax.broadcasted_iota(jnp.int32, (1, _B), 1)
    onehot = jnp.logical_and(bcol == gids, validc).astype(jnp.float32)
    cnt = jnp.sum(onehot, axis=0, keepdims=True)
    sums = lax.dot_general(onehot, x, (((0,), (0,)), ((), ())),
                           preferred_element_type=jnp.float32)
    cntc = cnt.reshape(_B, 1)
    mean = sums / jnp.maximum(cntc, 1.0)
    neg = jnp.float32(-3.0e38)
    mxs = []
    for g in range(_B):
        m = onehot[:, g:g + 1] > 0.0
        mxs.append(jnp.max(jnp.where(m, x, neg), axis=0, keepdims=True))
    mx = jnp.concatenate(mxs, axis=0)
    mx = jnp.where(cntc > 0.0, mx, 0.0)
    z = prev_ref[...] + jnp.concatenate([mx, mean], axis=1)
    if final:
        zz = jnp.maximum(
            jnp.dot(z, l1w_ref[...], preferred_element_type=jnp.float32)
            + l1b_ref[...], 0.0)
        out_ref[...] = jax.nn.sigmoid(
            jnp.dot(zz, l2w_ref[...], preferred_element_type=jnp.float32)
            + l2b_ref[...])
    else:
        out_ref[...] = z


def _tc_readout(x, bcol, prev, k):
    return pl.pallas_call(
        functools.partial(_tc_readout_body, k, False),
        out_shape=jax.ShapeDtypeStruct((_B, 2 * _D), jnp.float32),
    )(x, bcol, prev)


def _tc_readout_final(x, bcol, prev, k, l1w, l1b, l2w, l2b):
    return pl.pallas_call(
        functools.partial(_tc_readout_body, k, True),
        out_shape=jax.ShapeDtypeStruct((_B, 1), jnp.float32),
    )(x, bcol, prev, l1w, l1b, l2w, l2b)


_L = [
    dict(n=10000, n_pad=10240, k=5000, k_pad=5120),
    dict(n=5000, n_pad=5120, k=2500, k_pad=2560),
    dict(n=2500, n_pad=2560, k=1250, k_pad=1280),
]

_sc_deg1 = _make_sc_deg(_L[0]["n_pad"])
_AGG_PAD = _L[0]["n_pad"]
_sc_agg1 = _make_sc_agg()
_sc_pools = [_make_sc_pool(p["n"], p["n_pad"], p["k"], p["k_pad"], i < 2)
             for i, p in enumerate(_L)]


def kernel(x, pos, edge_index, edge_attr, strata_data, batch, W1, b1, W2, b2,
           W3, b3, ws1, ws2, ws3, lin1_w, lin1_b, lin2_w, lin2_b):
    p1, p2, p3 = _L
    src = jnp.concatenate(
        [edge_index[0], jnp.zeros((_EPAD - _E,), jnp.int32)]).reshape(_ER, 128)
    dst = jnp.concatenate(
        [edge_index[1], jnp.full((_EPAD - _E,), p1["n"], jnp.int32)]
    ).reshape(_ER, 128)
    h0 = jnp.pad(jnp.concatenate([x, pos], axis=1),
                 ((0, p1["n_pad"] - p1["n"]), (0, 0)))
    batch2d = jnp.pad(batch, (0, p1["n_pad"] - p1["n"])).reshape(-1, 128)

    ws = [ws1, ws2, ws3]
    Ws = [W1, W2, W3]
    bs = [b1.reshape(1, -1), b2.reshape(1, -1), b3.reshape(1, -1)]

    z = jnp.zeros((_B, 2 * _D), jnp.float32)
    h = h0
    degp, srcA, dstA, cntA, srcB, dstB, cntB = _sc_deg1(src, dst)
    elists = [(srcA, dstA, cntA), (srcB, dstB, cntB)]
    out = None
    for t, pt in enumerate(_L):
        g, dinv = _tc_prep(h, Ws[t], degp)
        gp = g if pt["n_pad"] == _AGG_PAD else jnp.pad(
            g, ((0, _AGG_PAD - pt["n_pad"]), (0, 0)))
        s_parts = [_sc_agg1(gp, es, ed, ec) for es, ed, ec in elists]
        hp, score = _tc_post_a(s_parts, g, dinv, bs[t], ws[t])
        keep2d, nid2d, gate2d = _tc_post_b(
            score.reshape(-1, 128), pt["n"], pt["k"])
        if t < 2:
            xnew, bnew, src, dst, cnt, degp = _sc_pools[t](
                hp, keep2d, nid2d, gate2d, batch2d, src, dst)
            elists = [(src, dst, cnt)]
        else:
            xnew, bnew = _sc_pools[t](hp, keep2d, nid2d, gate2d, batch2d)
        bcol = bnew.reshape(-1, 1)
        if t < 2:
            z = _tc_readout(xnew, bcol, z, pt["k"])
        else:
            out = _tc_readout_final(
                xnew, bcol, z, pt["k"], lin1_w, lin1_b.reshape(1, -1),
                lin2_w, lin2_b.reshape(1, -1))
        h = xnew
        batch2d = bnew
    return out
